# baseline (speedup 1.0000x reference)
import functools

import jax
import jax.numpy as jnp
from jax import lax
from jax.experimental import pallas as pl
from jax.experimental.pallas import tpu as pltpu
from jax.experimental.pallas import tpu_sc as plsc

N = 50000
E = 1600000
CH = 8
CONVS = 8
HID = 32
NT = 6
ENTRIES = 512

NP = 51200
NW = 32
CPW = 392
CHUNK = 128
KG = 8
MACRO = CPW // KG
EP = NW * CPW * CHUNK
NPW = NP // NW
RPT = NP // 16
NBLK = 16
F32 = jnp.float32
I32 = jnp.int32

_SC_PARAMS = pltpu.CompilerParams(use_tc_tiling_on_sc=False)


@functools.cache
def _sc_mesh():
    return plsc.VectorSubcoreMesh(core_axis_name="c", subcore_axis_name="s",
                                  num_cores=2, num_subcores=16)


def _worker_id():
    return lax.axis_index("s") * 2 + lax.axis_index("c")


def _sc_deg_body(dst_hbm, zeros_hbm, ones_hbm,
                 deg_out,
                 acc8, ones_v, didx, sem):
    cid = lax.axis_index("c")
    sid = lax.axis_index("s")
    wid = _worker_id()

    pltpu.sync_copy(zeros_hbm.at[pl.ds(sid * RPT, RPT)],
                    acc8.at[pl.ds(sid * RPT, RPT)])
    pltpu.sync_copy(ones_hbm, ones_v)
    plsc.subcore_barrier()

    @pl.loop(0, MACRO)
    def _deg(m):
        pltpu.sync_copy(dst_hbm.at[wid, pl.ds(m * KG, KG)], didx)
        descs = [
            pltpu.async_copy(ones_v, acc8.at[didx.at[j]], sem, add=True)
            for j in range(KG)
        ]
        for d in descs:
            d.wait()

    plsc.subcore_barrier()
    pltpu.sync_copy(acc8.at[pl.ds(sid * RPT, RPT)],
                    deg_out.at[cid, pl.ds(sid * RPT, RPT)])


def _sc_deg(dstp, zeros, ones128):
    return pl.kernel(
        _sc_deg_body,
        out_type=jax.ShapeDtypeStruct((2, NP, CH), F32),
        mesh=_sc_mesh(),
        compiler_params=_SC_PARAMS,
        scratch_types=[
            pltpu.VMEM_SHARED((NP, CH), F32),
            pltpu.VMEM((CHUNK, CH), F32),
            pltpu.VMEM((KG, CHUNK), I32),
            pltpu.SemaphoreType.DMA,
        ],
    )(dstp, zeros, ones128)


def _sc_prop_body(hwp_hbm, src_hbm, dst_hbm, zeros_hbm,
                  acc_out,
                  acc, sidx, didx, rows, gsem, ssem, sisem, disem):
    cid = lax.axis_index("c")
    sid = lax.axis_index("s")
    wid = _worker_id()

    sl = pl.ds(sid * RPT, RPT)

    @pl.when(cid == 0)
    def _():
        pltpu.sync_copy(hwp_hbm.at[sl], acc.at[sl])

    @pl.when(cid != 0)
    def _():
        pltpu.sync_copy(zeros_hbm.at[sl], acc.at[sl])

    plsc.subcore_barrier()

    def load_sidx(m, b):
        pltpu.async_copy(src_hbm.at[wid, pl.ds(m * KG, KG)], sidx.at[b],
                         sisem)

    def load_didx(m, b):
        pltpu.async_copy(dst_hbm.at[wid, pl.ds(m * KG, KG)], didx.at[b],
                         disem)

    def wait_idx(sem, bufs):
        pltpu.make_async_copy(src_hbm.at[wid, pl.ds(0, KG)], bufs.at[0],
                              sem).wait()

    def issue_gathers(b):
        for j in range(KG):
            pltpu.async_copy(hwp_hbm.at[sidx.at[b, j]],
                             rows.at[b, pl.ds(j * CHUNK, CHUNK)], gsem)

    def issue_scatters(b):
        for j in range(KG):
            pltpu.async_copy(rows.at[b, pl.ds(j * CHUNK, CHUNK)],
                             acc.at[didx.at[b, j]], ssem, add=True)

    def drain(sem, b):
        pltpu.make_async_copy(zeros_hbm.at[pl.ds(0, KG * CHUNK)],
                              rows.at[b], sem).wait()

    load_sidx(0, 0)
    load_didx(0, 0)
    wait_idx(sisem, sidx)
    wait_idx(disem, didx)
    issue_gathers(0)
    load_sidx(1, 1)
    load_didx(1, 1)
    drain(gsem, 0)
    load_sidx(2, 0)
    wait_idx(sisem, sidx)
    issue_gathers(1)
    issue_scatters(0)

    @pl.loop(0, (MACRO - 1) // 2)
    def _edges(p):
        for half in range(2):
            m = 2 * p + 1 + half
            b = 1 - half
            drain(ssem, 1 - b)
            @pl.when(m < MACRO - 1)
            def _():
                load_didx(m + 1, 1 - b)
            drain(gsem, b)
            @pl.when(m < MACRO - 2)
            def _():
                load_sidx(m + 2, b)
            @pl.when(m < MACRO - 1)
            def _():
                wait_idx(sisem, sidx)
                issue_gathers(1 - b)
            wait_idx(disem, didx)
            issue_scatters(b)

    drain(ssem, 0)

    plsc.subcore_barrier()
    pltpu.sync_copy(acc.at[sl], acc_out.at[cid, sl])


def _sc_prop(hwp, srcp, dstp, zeros):
    return pl.kernel(
        _sc_prop_body,
        out_type=jax.ShapeDtypeStruct((2, NP, CH), F32),
        mesh=_sc_mesh(),
        compiler_params=_SC_PARAMS,
        scratch_types=[
            pltpu.VMEM_SHARED((NP, CH), F32),
            pltpu.VMEM((2, KG, CHUNK), I32),
            pltpu.VMEM((2, KG, CHUNK), I32),
            pltpu.VMEM((2, KG * CHUNK, CH), F32),
            pltpu.SemaphoreType.DMA,
            pltpu.SemaphoreType.DMA,
            pltpu.SemaphoreType.DMA,
            pltpu.SemaphoreType.DMA,
        ],
    )(hwp, srcp, dstp, zeros)


def _sc_entry_body(h_hbm, eidx_hbm, out_hbm, eidx_v, rows, sem):
    wid = _worker_id()
    epw = ENTRIES // NW
    pltpu.sync_copy(eidx_hbm.at[pl.ds(wid * epw, epw)], eidx_v)
    pltpu.async_copy(h_hbm.at[eidx_v], rows, sem).wait()
    pltpu.sync_copy(rows, out_hbm.at[pl.ds(wid * epw, epw)])


def _sc_entry_gather(h, eidx):
    epw = ENTRIES // NW
    return pl.kernel(
        _sc_entry_body,
        out_type=jax.ShapeDtypeStruct((ENTRIES, CH), F32),
        mesh=_sc_mesh(),
        compiler_params=_SC_PARAMS,
        scratch_types=[
            pltpu.VMEM((epw,), I32),
            pltpu.VMEM((epw, CH), F32),
            pltpu.SemaphoreType.DMA,
        ],
    )(h, eidx)


def _row_mask(blk):
    rows = lax.broadcasted_iota(I32, (RPT, CH), 0) + blk * RPT
    return rows < N


def _bn_matmul(hnew, sums_ref, gamma_ref, beta_ref, w_ref, dinv_col, mask):
    mu = sums_ref[0:1, :] / N
    var = sums_ref[1:2, :] / N - mu * mu
    scale = gamma_ref[0:1, :] * lax.rsqrt(var + 1e-5)
    hb = (hnew - mu) * scale + beta_ref[0:1, :]
    hwp = jnp.dot(hb, w_ref[...], preferred_element_type=F32) * dinv_col
    return jnp.where(mask, hwp, 0.0)


def _stats_phases(p, mask, hnew, sums, hwp_out, apply_fn):
    @pl.when(p == 0)
    def _():
        sums[...] = jnp.zeros_like(sums)

    @pl.when(p < NBLK)
    def _():
        sums[0:1, :] += jnp.sum(hnew, axis=0, keepdims=True)
        sums[1:2, :] += jnp.sum(hnew * hnew, axis=0, keepdims=True)
        hwp_out[...] = jnp.zeros_like(hnew)

    @pl.when(p >= NBLK)
    def _():
        hwp_out[...] = apply_fn()


def _tc_b0_body(x_ref, emb_ref, d0_ref, d1_ref, gamma_ref, beta_ref, w_ref,
                h_out, hwp_out, dinv_out, sums):
    p = pl.program_id(0)
    blk = p % NBLK
    mask = _row_mask(blk)

    deg = d0_ref[:, 0:1] + d1_ref[:, 0:1] + 1.0
    dinv_col = lax.rsqrt(deg)
    dinv_out[...] = dinv_col
    onehot = (x_ref[...] ==
              lax.broadcasted_iota(I32, (1, NT), 1)).astype(F32)
    h0 = jnp.dot(onehot, emb_ref[...], preferred_element_type=F32)
    hnew = jnp.where(mask, h0, 0.0)
    h_out[...] = hnew

    _stats_phases(p, mask, hnew, sums, hwp_out,
                  lambda: _bn_matmul(hnew, sums, gamma_ref, beta_ref, w_ref,
                                     dinv_col, mask))


def _tc_b0(xp2, emb, deg_parts, gamma, beta, w):
    blk = lambda i: (i % NBLK, 0)
    zero = lambda i: (0, 0)
    return pl.pallas_call(
        _tc_b0_body,
        grid=(2 * NBLK,),
        in_specs=[
            pl.BlockSpec((RPT, 1), blk),
            pl.BlockSpec((NT, CH), zero),
            pl.BlockSpec((RPT, CH), blk),
            pl.BlockSpec((RPT, CH), blk),
            pl.BlockSpec((1, CH), zero),
            pl.BlockSpec((1, CH), zero),
            pl.BlockSpec((CH, CH), zero),
        ],
        out_specs=[
            pl.BlockSpec((RPT, CH), blk),
            pl.BlockSpec((RPT, CH), blk),
            pl.BlockSpec((RPT, 1), blk),
        ],
        out_shape=[
            jax.ShapeDtypeStruct((NP, CH), F32),
            jax.ShapeDtypeStruct((NP, CH), F32),
            jax.ShapeDtypeStruct((NP, 1), F32),
        ],
        scratch_shapes=[pltpu.VMEM((2, CH), F32)],
    )(xp2, emb, deg_parts[0], deg_parts[1], gamma, beta, w)


def _tc_b_body(h_ref, a0_ref, a1_ref, dinv_ref, b_ref,
               gamma_ref, beta_ref, w_ref,
               h_out, hwp_out, sums):
    p = pl.program_id(0)
    blk = p % NBLK
    mask = _row_mask(blk)
    dinv_col = dinv_ref[...]

    conv = (a0_ref[...] + a1_ref[...]) * dinv_col + b_ref[0:1, :]
    hnew = jnp.where(mask, jax.nn.relu(h_ref[...] + conv), 0.0)
    h_out[...] = hnew

    _stats_phases(p, mask, hnew, sums, hwp_out,
                  lambda: _bn_matmul(hnew, sums, gamma_ref, beta_ref, w_ref,
                                     dinv_col, mask))


def _tc_b(h, acc_parts, dinv, b, gamma, beta, w):
    blk = lambda i: (i % NBLK, 0)
    zero = lambda i: (0, 0)
    return pl.pallas_call(
        _tc_b_body,
        grid=(2 * NBLK,),
        in_specs=[
            pl.BlockSpec((RPT, CH), blk),
            pl.BlockSpec((RPT, CH), blk),
            pl.BlockSpec((RPT, CH), blk),
            pl.BlockSpec((RPT, 1), blk),
            pl.BlockSpec((1, CH), zero),
            pl.BlockSpec((1, CH), zero),
            pl.BlockSpec((1, CH), zero),
            pl.BlockSpec((CH, CH), zero),
        ],
        out_specs=[
            pl.BlockSpec((RPT, CH), blk),
            pl.BlockSpec((RPT, CH), blk),
        ],
        out_shape=[
            jax.ShapeDtypeStruct((NP, CH), F32),
            jax.ShapeDtypeStruct((NP, CH), F32),
        ],
        scratch_shapes=[pltpu.VMEM((2, CH), F32)],
    )(h, acc_parts[0], acc_parts[1], dinv, b, gamma, beta, w)


def _tc_fin_body(h_ref, a0_ref, a1_ref, dinv_ref, b_ref, h_out):
    blk = pl.program_id(0)
    mask = _row_mask(blk)
    conv = (a0_ref[...] + a1_ref[...]) * dinv_ref[...] + b_ref[0:1, :]
    h_out[...] = jnp.where(mask, jax.nn.relu(h_ref[...] + conv), 0.0)


def _tc_fin(h, acc_parts, dinv, b):
    blk = lambda i: (i, 0)
    zero = lambda i: (0, 0)
    return pl.pallas_call(
        _tc_fin_body,
        grid=(NBLK,),
        in_specs=[
            pl.BlockSpec((RPT, CH), blk),
            pl.BlockSpec((RPT, CH), blk),
            pl.BlockSpec((RPT, CH), blk),
            pl.BlockSpec((RPT, 1), blk),
            pl.BlockSpec((1, CH), zero),
        ],
        out_specs=pl.BlockSpec((RPT, CH), blk),
        out_shape=jax.ShapeDtypeStruct((NP, CH), F32),
    )(h, acc_parts[0], acc_parts[1], dinv, b)


def _tc_head_body(xe_ref, hw_ref, hb_ref, ow_ref, ob_ref, out_ref):
    xe = xe_ref[...]
    u = jnp.dot(xe, hw_ref[0:CH, :], preferred_element_type=F32)
    v = jnp.dot(xe, hw_ref[CH:2 * CH, :], preferred_element_type=F32)
    vt = v.T
    hb = hb_ref[...]
    ow = ow_ref[...]
    acc = jnp.zeros((ENTRIES, ENTRIES), F32)
    for k in range(HID):
        term = u[:, k:k + 1] + vt[k:k + 1, :] + hb[0, k]
        acc = acc + jax.nn.relu(term) * ow[k, 0]
    out_ref[...] = acc + ob_ref[0, 0]


def _tc_head(xe, hid_W, hid_b, out_W, out_b):
    zero2 = lambda: (0, 0)
    return pl.pallas_call(
        _tc_head_body,
        grid=(1,),
        in_specs=[
            pl.BlockSpec((ENTRIES, CH), lambda i: (0, 0)),
            pl.BlockSpec((2 * CH, HID), lambda i: (0, 0)),
            pl.BlockSpec((1, HID), lambda i: (0, 0)),
            pl.BlockSpec((HID, 1), lambda i: (0, 0)),
            pl.BlockSpec((1, 1), lambda i: (0, 0)),
        ],
        out_specs=pl.BlockSpec((ENTRIES, ENTRIES), lambda i: (0, 0)),
        out_shape=jax.ShapeDtypeStruct((ENTRIES, ENTRIES), F32),
    )(xe, hid_W, hid_b, out_W, out_b)


def kernel(x, edge_index, entry_index, emb, bn_gamma, bn_beta, conv_W, conv_b,
           hid_W, hid_b, out_W, out_b):
    pad_e = EP - E
    pad_idx = (N + jnp.arange(pad_e, dtype=I32) % (NP - N)).astype(I32)
    srcp = jnp.concatenate([edge_index[0], pad_idx]).reshape(NW, CPW, CHUNK)
    dstp = jnp.concatenate([edge_index[1], pad_idx]).reshape(NW, CPW, CHUNK)
    xp2 = jnp.concatenate(
        [x, jnp.zeros((NP - N,), I32)]).reshape(NP, 1)
    zeros = jnp.zeros((NP, CH), F32)
    ones128 = jnp.ones((CHUNK, CH), F32)

    deg_parts = _sc_deg(dstp, zeros, ones128)

    h, hwp, dinv = _tc_b0(xp2, emb, deg_parts,
                          bn_gamma[0:1, :], bn_beta[0:1, :], conv_W[0])

    for i in range(CONVS):
        acc_parts = _sc_prop(hwp, srcp, dstp, zeros)
        if i < CONVS - 1:
            h, hwp = _tc_b(h, acc_parts, dinv, conv_b[i:i + 1, :],
                           bn_gamma[i + 1:i + 2, :], bn_beta[i + 1:i + 2, :],
                           conv_W[i + 1])
        else:
            h = _tc_fin(h, acc_parts, dinv, conv_b[i:i + 1, :])

    xe = _sc_entry_gather(h, entry_index)

    out = _tc_head(xe, hid_W,
                   hid_b.reshape(1, HID), out_W, out_b.reshape(1, 1))
    return out


# speedup vs baseline: 1.1812x; 1.1812x over previous
import functools

import jax
import jax.numpy as jnp
from jax import lax
from jax.experimental import pallas as pl
from jax.experimental.pallas import tpu as pltpu
from jax.experimental.pallas import tpu_sc as plsc

N = 50000
E = 1600000
CH = 8
CONVS = 8
HID = 32
NT = 6
ENTRIES = 512

NP = 51200
NW = 32
CPW = 392
CHUNK = 128
KG = 8
MACRO = CPW // KG
EP = NW * CPW * CHUNK
NPW = NP // NW
RPT = NP // 16
NBLK = 16
F32 = jnp.float32
I32 = jnp.int32

_SC_PARAMS = pltpu.CompilerParams(use_tc_tiling_on_sc=False)


@functools.cache
def _sc_mesh():
    return plsc.VectorSubcoreMesh(core_axis_name="c", subcore_axis_name="s",
                                  num_cores=2, num_subcores=16)


def _worker_id():
    return lax.axis_index("s") * 2 + lax.axis_index("c")


def _sc_deg_body(dst_hbm, zeros_hbm, ones_hbm,
                 deg_out,
                 acc8, ones_v, didx, drows, sem, isem):
    cid = lax.axis_index("c")
    sid = lax.axis_index("s")
    wid = _worker_id()

    pltpu.sync_copy(zeros_hbm.at[pl.ds(sid * RPT, RPT)],
                    acc8.at[pl.ds(sid * RPT, RPT)])
    pltpu.sync_copy(ones_hbm, ones_v)
    plsc.subcore_barrier()

    def load_didx(m, b):
        pltpu.async_copy(dst_hbm.at[wid, pl.ds(m * KG, KG)], didx.at[b],
                         isem)

    def wait_idx():
        pltpu.make_async_copy(dst_hbm.at[wid, pl.ds(0, KG)], didx.at[0],
                              isem).wait()

    def issue_scatters(b):
        for j in range(KG):
            pltpu.async_copy(ones_v, acc8.at[didx.at[b, j]], sem, add=True)

    def drain_scatters():
        pltpu.make_async_copy(zeros_hbm.at[pl.ds(0, KG * CHUNK)], drows,
                              sem).wait()

    load_didx(0, 0)
    load_didx(1, 1)
    wait_idx()
    issue_scatters(0)

    @pl.loop(0, (MACRO - 1) // 2)
    def _deg(p):
        for half in range(2):
            m = 2 * p + 1 + half
            b = 1 - half
            drain_scatters()
            @pl.when(m < MACRO - 1)
            def _():
                load_didx(m + 1, 1 - b)
            wait_idx()
            issue_scatters(b)

    drain_scatters()

    plsc.subcore_barrier()
    pltpu.sync_copy(acc8.at[pl.ds(sid * RPT, RPT)],
                    deg_out.at[cid, pl.ds(sid * RPT, RPT)])


def _sc_deg(dstp, zeros, ones128):
    return pl.kernel(
        _sc_deg_body,
        out_type=jax.ShapeDtypeStruct((2, NP, CH), F32),
        mesh=_sc_mesh(),
        compiler_params=_SC_PARAMS,
        scratch_types=[
            pltpu.VMEM_SHARED((NP, CH), F32),
            pltpu.VMEM((CHUNK, CH), F32),
            pltpu.VMEM((2, KG, CHUNK), I32),
            pltpu.VMEM((KG * CHUNK, CH), F32),
            pltpu.SemaphoreType.DMA,
            pltpu.SemaphoreType.DMA,
        ],
    )(dstp, zeros, ones128)


def _sc_prop_body(hwp_hbm, src_hbm, dst_hbm, zeros_hbm,
                  acc_out,
                  table, acc, sidx, didx, rows, gsem, ssem, sisem, disem):
    cid = lax.axis_index("c")
    sid = lax.axis_index("s")
    wid = _worker_id()

    sl = pl.ds(sid * RPT, RPT)
    pltpu.sync_copy(hwp_hbm.at[sl], table.at[sl])

    @pl.when(cid == 0)
    def _():
        pltpu.sync_copy(hwp_hbm.at[sl], acc.at[sl])

    @pl.when(cid != 0)
    def _():
        pltpu.sync_copy(zeros_hbm.at[sl], acc.at[sl])

    plsc.subcore_barrier()

    def load_sidx(m, b):
        pltpu.async_copy(src_hbm.at[wid, pl.ds(m * KG, KG)], sidx.at[b],
                         sisem)

    def load_didx(m, b):
        pltpu.async_copy(dst_hbm.at[wid, pl.ds(m * KG, KG)], didx.at[b],
                         disem)

    def wait_idx(sem, bufs):
        pltpu.make_async_copy(src_hbm.at[wid, pl.ds(0, KG)], bufs.at[0],
                              sem).wait()

    def issue_gathers(b):
        for j in range(KG):
            pltpu.async_copy(table.at[sidx.at[b, j]],
                             rows.at[b, pl.ds(j * CHUNK, CHUNK)], gsem)

    def issue_scatters(b):
        for j in range(KG):
            pltpu.async_copy(rows.at[b, pl.ds(j * CHUNK, CHUNK)],
                             acc.at[didx.at[b, j]], ssem, add=True)

    def drain(sem, b):
        pltpu.make_async_copy(zeros_hbm.at[pl.ds(0, KG * CHUNK)],
                              rows.at[b], sem).wait()

    load_sidx(0, 0)
    load_didx(0, 0)
    wait_idx(sisem, sidx)
    wait_idx(disem, didx)
    issue_gathers(0)
    load_sidx(1, 1)
    load_didx(1, 1)
    drain(gsem, 0)
    load_sidx(2, 0)
    wait_idx(sisem, sidx)
    issue_gathers(1)
    issue_scatters(0)

    @pl.loop(0, (MACRO - 1) // 2)
    def _edges(p):
        for half in range(2):
            m = 2 * p + 1 + half
            b = 1 - half
            drain(ssem, 1 - b)
            @pl.when(m < MACRO - 1)
            def _():
                load_didx(m + 1, 1 - b)
            drain(gsem, b)
            @pl.when(m < MACRO - 2)
            def _():
                load_sidx(m + 2, b)
            @pl.when(m < MACRO - 1)
            def _():
                wait_idx(sisem, sidx)
                issue_gathers(1 - b)
            wait_idx(disem, didx)
            issue_scatters(b)

    drain(ssem, 0)

    plsc.subcore_barrier()
    pltpu.sync_copy(acc.at[sl], acc_out.at[cid, sl])


def _sc_prop(hwp, srcp, dstp, zeros):
    return pl.kernel(
        _sc_prop_body,
        out_type=jax.ShapeDtypeStruct((2, NP, CH), F32),
        mesh=_sc_mesh(),
        compiler_params=_SC_PARAMS,
        scratch_types=[
            pltpu.VMEM_SHARED((NP, CH), F32),
            pltpu.VMEM_SHARED((NP, CH), F32),
            pltpu.VMEM((2, KG, CHUNK), I32),
            pltpu.VMEM((2, KG, CHUNK), I32),
            pltpu.VMEM((2, KG * CHUNK, CH), F32),
            pltpu.SemaphoreType.DMA,
            pltpu.SemaphoreType.DMA,
            pltpu.SemaphoreType.DMA,
            pltpu.SemaphoreType.DMA,
        ],
    )(hwp, srcp, dstp, zeros)


def _sc_entry_body(h_hbm, eidx_hbm, out_hbm, eidx_v, rows, sem):
    wid = _worker_id()
    epw = ENTRIES // NW
    pltpu.sync_copy(eidx_hbm.at[pl.ds(wid * epw, epw)], eidx_v)
    pltpu.async_copy(h_hbm.at[eidx_v], rows, sem).wait()
    pltpu.sync_copy(rows, out_hbm.at[pl.ds(wid * epw, epw)])


def _sc_entry_gather(h, eidx):
    epw = ENTRIES // NW
    return pl.kernel(
        _sc_entry_body,
        out_type=jax.ShapeDtypeStruct((ENTRIES, CH), F32),
        mesh=_sc_mesh(),
        compiler_params=_SC_PARAMS,
        scratch_types=[
            pltpu.VMEM((epw,), I32),
            pltpu.VMEM((epw, CH), F32),
            pltpu.SemaphoreType.DMA,
        ],
    )(h, eidx)


def _row_mask(blk):
    rows = lax.broadcasted_iota(I32, (RPT, CH), 0) + blk * RPT
    return rows < N


def _bn_matmul(hnew, sums_ref, gamma_ref, beta_ref, w_ref, dinv_col, mask):
    mu = sums_ref[0:1, :] / N
    var = sums_ref[1:2, :] / N - mu * mu
    scale = gamma_ref[0:1, :] * lax.rsqrt(var + 1e-5)
    hb = (hnew - mu) * scale + beta_ref[0:1, :]
    hwp = jnp.dot(hb, w_ref[...], preferred_element_type=F32) * dinv_col
    return jnp.where(mask, hwp, 0.0)


def _stats_phases(p, mask, hnew, sums, hwp_out, apply_fn):
    @pl.when(p == 0)
    def _():
        sums[...] = jnp.zeros_like(sums)

    @pl.when(p < NBLK)
    def _():
        sums[0:1, :] += jnp.sum(hnew, axis=0, keepdims=True)
        sums[1:2, :] += jnp.sum(hnew * hnew, axis=0, keepdims=True)
        hwp_out[...] = jnp.zeros_like(hnew)

    @pl.when(p >= NBLK)
    def _():
        hwp_out[...] = apply_fn()


def _tc_b0_body(x_ref, emb_ref, d0_ref, d1_ref, gamma_ref, beta_ref, w_ref,
                h_out, hwp_out, dinv_out, sums):
    p = pl.program_id(0)
    blk = p % NBLK
    mask = _row_mask(blk)

    deg = d0_ref[:, 0:1] + d1_ref[:, 0:1] + 1.0
    dinv_col = lax.rsqrt(deg)
    dinv_out[...] = dinv_col
    onehot = (x_ref[...] ==
              lax.broadcasted_iota(I32, (1, NT), 1)).astype(F32)
    h0 = jnp.dot(onehot, emb_ref[...], preferred_element_type=F32)
    hnew = jnp.where(mask, h0, 0.0)
    h_out[...] = hnew

    _stats_phases(p, mask, hnew, sums, hwp_out,
                  lambda: _bn_matmul(hnew, sums, gamma_ref, beta_ref, w_ref,
                                     dinv_col, mask))


def _tc_b0(xp2, emb, deg_parts, gamma, beta, w):
    blk = lambda i: (i % NBLK, 0)
    zero = lambda i: (0, 0)
    return pl.pallas_call(
        _tc_b0_body,
        grid=(2 * NBLK,),
        in_specs=[
            pl.BlockSpec((RPT, 1), blk),
            pl.BlockSpec((NT, CH), zero),
            pl.BlockSpec((RPT, CH), blk),
            pl.BlockSpec((RPT, CH), blk),
            pl.BlockSpec((1, CH), zero),
            pl.BlockSpec((1, CH), zero),
            pl.BlockSpec((CH, CH), zero),
        ],
        out_specs=[
            pl.BlockSpec((RPT, CH), blk),
            pl.BlockSpec((RPT, CH), blk),
            pl.BlockSpec((RPT, 1), blk),
        ],
        out_shape=[
            jax.ShapeDtypeStruct((NP, CH), F32),
            jax.ShapeDtypeStruct((NP, CH), F32),
            jax.ShapeDtypeStruct((NP, 1), F32),
        ],
        scratch_shapes=[pltpu.VMEM((2, CH), F32)],
    )(xp2, emb, deg_parts[0], deg_parts[1], gamma, beta, w)


def _tc_b_body(h_ref, a0_ref, a1_ref, dinv_ref, b_ref,
               gamma_ref, beta_ref, w_ref,
               h_out, hwp_out, sums):
    p = pl.program_id(0)
    blk = p % NBLK
    mask = _row_mask(blk)
    dinv_col = dinv_ref[...]

    conv = (a0_ref[...] + a1_ref[...]) * dinv_col + b_ref[0:1, :]
    hnew = jnp.where(mask, jax.nn.relu(h_ref[...] + conv), 0.0)
    h_out[...] = hnew

    _stats_phases(p, mask, hnew, sums, hwp_out,
                  lambda: _bn_matmul(hnew, sums, gamma_ref, beta_ref, w_ref,
                                     dinv_col, mask))


def _tc_b(h, acc_parts, dinv, b, gamma, beta, w):
    blk = lambda i: (i % NBLK, 0)
    zero = lambda i: (0, 0)
    return pl.pallas_call(
        _tc_b_body,
        grid=(2 * NBLK,),
        in_specs=[
            pl.BlockSpec((RPT, CH), blk),
            pl.BlockSpec((RPT, CH), blk),
            pl.BlockSpec((RPT, CH), blk),
            pl.BlockSpec((RPT, 1), blk),
            pl.BlockSpec((1, CH), zero),
            pl.BlockSpec((1, CH), zero),
            pl.BlockSpec((1, CH), zero),
            pl.BlockSpec((CH, CH), zero),
        ],
        out_specs=[
            pl.BlockSpec((RPT, CH), blk),
            pl.BlockSpec((RPT, CH), blk),
        ],
        out_shape=[
            jax.ShapeDtypeStruct((NP, CH), F32),
            jax.ShapeDtypeStruct((NP, CH), F32),
        ],
        scratch_shapes=[pltpu.VMEM((2, CH), F32)],
    )(h, acc_parts[0], acc_parts[1], dinv, b, gamma, beta, w)


def _tc_fin_body(h_ref, a0_ref, a1_ref, dinv_ref, b_ref, h_out):
    blk = pl.program_id(0)
    mask = _row_mask(blk)
    conv = (a0_ref[...] + a1_ref[...]) * dinv_ref[...] + b_ref[0:1, :]
    h_out[...] = jnp.where(mask, jax.nn.relu(h_ref[...] + conv), 0.0)


def _tc_fin(h, acc_parts, dinv, b):
    blk = lambda i: (i, 0)
    zero = lambda i: (0, 0)
    return pl.pallas_call(
        _tc_fin_body,
        grid=(NBLK,),
        in_specs=[
            pl.BlockSpec((RPT, CH), blk),
            pl.BlockSpec((RPT, CH), blk),
            pl.BlockSpec((RPT, CH), blk),
            pl.BlockSpec((RPT, 1), blk),
            pl.BlockSpec((1, CH), zero),
        ],
        out_specs=pl.BlockSpec((RPT, CH), blk),
        out_shape=jax.ShapeDtypeStruct((NP, CH), F32),
    )(h, acc_parts[0], acc_parts[1], dinv, b)


def _tc_head_body(xe_ref, hw_ref, hb_ref, ow_ref, ob_ref, out_ref):
    xe = xe_ref[...]
    u = jnp.dot(xe, hw_ref[0:CH, :], preferred_element_type=F32)
    v = jnp.dot(xe, hw_ref[CH:2 * CH, :], preferred_element_type=F32)
    vt = v.T
    hb = hb_ref[...]
    ow = ow_ref[...]
    acc = jnp.zeros((ENTRIES, ENTRIES), F32)
    for k in range(HID):
        term = u[:, k:k + 1] + vt[k:k + 1, :] + hb[0, k]
        acc = acc + jax.nn.relu(term) * ow[k, 0]
    out_ref[...] = acc + ob_ref[0, 0]


def _tc_head(xe, hid_W, hid_b, out_W, out_b):
    zero2 = lambda: (0, 0)
    return pl.pallas_call(
        _tc_head_body,
        grid=(1,),
        in_specs=[
            pl.BlockSpec((ENTRIES, CH), lambda i: (0, 0)),
            pl.BlockSpec((2 * CH, HID), lambda i: (0, 0)),
            pl.BlockSpec((1, HID), lambda i: (0, 0)),
            pl.BlockSpec((HID, 1), lambda i: (0, 0)),
            pl.BlockSpec((1, 1), lambda i: (0, 0)),
        ],
        out_specs=pl.BlockSpec((ENTRIES, ENTRIES), lambda i: (0, 0)),
        out_shape=jax.ShapeDtypeStruct((ENTRIES, ENTRIES), F32),
    )(xe, hid_W, hid_b, out_W, out_b)


def kernel(x, edge_index, entry_index, emb, bn_gamma, bn_beta, conv_W, conv_b,
           hid_W, hid_b, out_W, out_b):
    pad_e = EP - E
    pad_idx = (N + jnp.arange(pad_e, dtype=I32) % (NP - N)).astype(I32)
    srcp = jnp.concatenate([edge_index[0], pad_idx]).reshape(NW, CPW, CHUNK)
    dstp = jnp.concatenate([edge_index[1], pad_idx]).reshape(NW, CPW, CHUNK)
    xp2 = jnp.concatenate(
        [x, jnp.zeros((NP - N,), I32)]).reshape(NP, 1)
    zeros = jnp.zeros((NP, CH), F32)
    ones128 = jnp.ones((CHUNK, CH), F32)

    deg_parts = _sc_deg(dstp, zeros, ones128)

    h, hwp, dinv = _tc_b0(xp2, emb, deg_parts,
                          bn_gamma[0:1, :], bn_beta[0:1, :], conv_W[0])

    for i in range(CONVS):
        acc_parts = _sc_prop(hwp, srcp, dstp, zeros)
        if i < CONVS - 1:
            h, hwp = _tc_b(h, acc_parts, dinv, conv_b[i:i + 1, :],
                           bn_gamma[i + 1:i + 2, :], bn_beta[i + 1:i + 2, :],
                           conv_W[i + 1])
        else:
            h = _tc_fin(h, acc_parts, dinv, conv_b[i:i + 1, :])

    xe = _sc_entry_gather(h, entry_index)

    out = _tc_head(xe, hid_W,
                   hid_b.reshape(1, HID), out_W, out_b.reshape(1, 1))
    return out
